# baseline (speedup 1.0000x reference)
import functools
from math import sqrt

import jax
import jax.numpy as jnp
from jax.experimental import pallas as pl
from jax.experimental.pallas import tpu as pltpu

LOCAL_HALF = 8
STRIDE_P1 = 65


def _diags(L):
    return [d for d in range(L) if d <= LOCAL_HALF or d % STRIDE_P1 == 0]


def _body(q_ref, k_ref, v_ref, o_ref, c_ref, *, T, D, PAD, diags, scale):
    qi = pl.program_id(1)
    i0 = qi * T

    @pl.when(qi == 0)
    def _():
        c_ref[...] = jnp.zeros_like(c_ref)

    qb = q_ref[0]
    rows = jax.lax.broadcasted_iota(jnp.int32, (T, 1), 0) + i0

    num = jnp.zeros((T, D), jnp.float32)
    den = jnp.zeros((T, 1), jnp.float32)
    for d in diags:
        ks = k_ref[0, pl.ds(i0 - d + PAD, T), :]
        s = jnp.sum(qb * ks, axis=1, keepdims=True)
        w = jnp.where(rows >= d, jnp.exp(scale * s) - 1.0, 0.0)
        vs = v_ref[0, pl.ds(i0 - d + PAD, T), :]
        num = num + w * vs
        den = den + w

    vb = v_ref[0, pl.ds(i0 + PAD, T), :]
    ri = jax.lax.broadcasted_iota(jnp.int32, (T, T), 0)
    ci = jax.lax.broadcasted_iota(jnp.int32, (T, T), 1)
    tril = (ri >= ci).astype(jnp.float32)
    p = jax.lax.dot(tril, vb, preferred_element_type=jnp.float32)
    carry = c_ref[...]
    c_ref[...] = carry + jnp.sum(vb, axis=0, keepdims=True)

    o_ref[0] = (carry + p + num) / (
        (rows + 1).astype(jnp.float32) + den)


def kernel(queries, keys, values, attn_mask):
    B, L, H, D = queries.shape
    del attn_mask
    scale = 1.0 / sqrt(D)
    T = 256
    PAD = L
    diags = tuple(_diags(L))

    qt = jnp.transpose(queries, (0, 2, 1, 3)).reshape(B * H, L, D)
    kt = jnp.transpose(keys, (0, 2, 1, 3)).reshape(B * H, L, D)
    vt = jnp.transpose(values, (0, 2, 1, 3)).reshape(B * H, L, D)
    zpad = jnp.zeros((B * H, PAD, D), jnp.float32)
    kp = jnp.concatenate([zpad, kt], axis=1)
    vp = jnp.concatenate([zpad, vt], axis=1)

    body = functools.partial(_body, T=T, D=D, PAD=PAD, diags=diags,
                             scale=scale)
    out = pl.pallas_call(
        body,
        grid=(B * H, L // T),
        in_specs=[
            pl.BlockSpec((1, T, D), lambda bh, qi: (bh, qi, 0)),
            pl.BlockSpec((1, PAD + L, D), lambda bh, qi: (bh, 0, 0)),
            pl.BlockSpec((1, PAD + L, D), lambda bh, qi: (bh, 0, 0)),
        ],
        out_specs=pl.BlockSpec((1, T, D), lambda bh, qi: (bh, qi, 0)),
        out_shape=jax.ShapeDtypeStruct((B * H, L, D), jnp.float32),
        scratch_shapes=[pltpu.VMEM((1, D), jnp.float32)],
    )(qt, kp, vp)
    return jnp.transpose(out.reshape(B, H, L, D), (0, 2, 1, 3))
